# baseline (speedup 1.0000x reference)
import functools

import jax
import jax.numpy as jnp
from jax import lax
from jax.experimental import pallas as pl
from jax.experimental.pallas import tpu as pltpu
from jax.experimental.pallas import tpu_sc as plsc

NC = 2
NS = 16
NW = NC * NS
NB = 2
LW = 128
CR = 40


def _sc_bag(pidx, didx, zeros, table2, batch):
    rtot, lw = pidx.shape
    d2 = table2.shape[1]
    d = d2 // 2
    rpw = rtot // NW
    bpw = batch // NW
    apw = 2 * bpw
    mesh = plsc.VectorSubcoreMesh(core_axis_name="c", subcore_axis_name="s")

    @functools.partial(
        pl.kernel,
        out_type=jax.ShapeDtypeStruct((batch, d2), jnp.float32),
        mesh=mesh,
        scratch_types=[
            pltpu.VMEM((CR, lw), jnp.int32),
            pltpu.VMEM((CR, lw), jnp.int32),
            [pltpu.VMEM((lw, d2), jnp.float32) for _ in range(NB)],
            pltpu.VMEM((apw // 2, d2), jnp.float32),
            pltpu.VMEM((bpw // 2, d2), jnp.float32),
            pltpu.VMEM_SHARED((NS * 2 * bpw, d2), jnp.float32),
            [pltpu.SemaphoreType.DMA for _ in range(NB)],
            [pltpu.SemaphoreType.DMA for _ in range(NB)],
        ],
        compiler_params=pltpu.CompilerParams(use_tc_tiling_on_sc=False),
    )
    def k(pidx_h, didx_h, zeros_h, table_h, out_h,
          xblk, dblk, gbs, obuf, sbuf, acc, gsems, ssems):
        c = lax.axis_index("c")
        s = lax.axis_index("s")
        wid = s * NC + c
        rbase = wid * rpw
        abase = s * apw

        pltpu.sync_copy(zeros_h, obuf)
        pltpu.sync_copy(obuf, acc.at[pl.ds(abase, apw // 2)])
        pltpu.sync_copy(obuf, acc.at[pl.ds(abase + apw // 2, apw // 2)])

        def chunk(ic, carry):
            pltpu.sync_copy(pidx_h.at[pl.ds(rbase + ic * CR, CR)], xblk)
            pltpu.sync_copy(didx_h.at[pl.ds(rbase + ic * CR, CR)], dblk)
            for b in range(NB):
                pltpu.async_copy(table_h.at[xblk.at[b]], gbs[b], gsems[b])

            def body(i, carry2):
                g = NB * i
                for b in range(NB):
                    pltpu.make_async_copy(table_h.at[xblk.at[0]], gbs[b],
                                          gsems[b]).wait()
                    pltpu.async_copy(gbs[b], acc.at[dblk.at[g + b]], ssems[b],
                                     add=True)
                for b in range(NB):
                    pltpu.make_async_copy(gbs[b], acc.at[dblk.at[0]],
                                          ssems[b]).wait()
                    pltpu.async_copy(table_h.at[xblk.at[g + NB + b]], gbs[b],
                                     gsems[b])
                return carry2

            lax.fori_loop(0, CR // NB - 1, body, 0)

            g_last = CR - NB
            for b in range(NB):
                pltpu.make_async_copy(table_h.at[xblk.at[0]], gbs[b],
                                      gsems[b]).wait()
                pltpu.async_copy(gbs[b], acc.at[dblk.at[g_last + b]], ssems[b],
                                 add=True)
            for b in range(NB):
                pltpu.make_async_copy(gbs[b], acc.at[dblk.at[0]],
                                      ssems[b]).wait()
            return carry

        lax.fori_loop(0, rpw // CR, chunk, 0)

        for h in range(2):
            pltpu.sync_copy(acc.at[pl.ds(abase + h * (apw // 2), apw // 2)],
                            obuf)

            def crow(j, carry):
                for cc in range(d // 16):
                    lo = obuf[2 * j, pl.ds(cc * 16, 16)]
                    hi = obuf[2 * j + 1, pl.ds(d + cc * 16, 16)]
                    sbuf[j, pl.ds(cc * 16, 16)] = lo + hi
                return carry

            lax.fori_loop(0, bpw // 2, crow, 0)
            pltpu.sync_copy(
                sbuf, out_h.at[pl.ds(wid * bpw + h * (bpw // 2), bpw // 2)])

    return k(pidx, didx, zeros, table2)


def _tc_mlp(s, W1, b1, W2, b2, n_rows):
    batch = s.shape[0]
    d = W1.shape[1]
    hid = W1.shape[0]
    ncls = W2.shape[0]
    hp = 128
    W1p = jnp.zeros((hp, d), jnp.float32).at[:hid].set(W1)
    b1p = jnp.zeros((1, hp), jnp.float32).at[0, :hid].set(b1)
    W2p = jnp.zeros((ncls, hp), jnp.float32).at[:, :hid].set(W2)
    b2p = b2.reshape(1, ncls)
    bt = 512
    scale = 1.0 / n_rows

    def body(s_ref, w1_ref, b1_ref, w2_ref, b2_ref, o_ref):
        m = s_ref[...][:, :d] * scale
        h = lax.dot_general(m, w1_ref[...], (((1,), (1,)), ((), ())),
                            preferred_element_type=jnp.float32)
        h = jnp.maximum(h + b1_ref[...], 0.0)
        o = lax.dot_general(h, w2_ref[...], (((1,), (1,)), ((), ())),
                            preferred_element_type=jnp.float32)
        o_ref[...] = o + b2_ref[...]

    return pl.pallas_call(
        body,
        grid=(batch // bt,),
        in_specs=[
            pl.BlockSpec((bt, 2 * d), lambda i: (i, 0)),
            pl.BlockSpec((hp, d), lambda i: (0, 0)),
            pl.BlockSpec((1, hp), lambda i: (0, 0)),
            pl.BlockSpec((ncls, hp), lambda i: (0, 0)),
            pl.BlockSpec((1, ncls), lambda i: (0, 0)),
        ],
        out_specs=pl.BlockSpec((bt, ncls), lambda i: (i, 0)),
        out_shape=jax.ShapeDtypeStruct((batch, ncls), jnp.float32),
    )(s, W1p, b1p, W2p, b2p)


def kernel(x, embed, W1, b1, W2, b2):
    batch, hist = x.shape
    vocab, d = embed.shape
    n = batch * hist
    table2 = embed.reshape(vocab // 2, 2 * d)
    xflat = x.reshape(n)
    pidx = lax.shift_right_logical(xflat, 1).reshape(-1, LW)
    bpw = batch // NW
    i = jnp.arange(n, dtype=jnp.int32)
    w = i // (hist * bpw)
    sub = w // NC
    brow_local = (i // hist) % bpw
    didx = (sub * (2 * bpw) + 2 * brow_local + (xflat & 1)).reshape(-1, LW)
    zeros = jnp.zeros((bpw, 2 * d), jnp.float32)
    sums2 = _sc_bag(pidx, didx, zeros, table2, batch)
    return _tc_mlp(sums2, W1, b1, W2, b2, batch)


# speedup vs baseline: 1.2529x; 1.2529x over previous
import functools

import jax
import jax.numpy as jnp
from jax import lax
from jax.experimental import pallas as pl
from jax.experimental.pallas import tpu as pltpu
from jax.experimental.pallas import tpu_sc as plsc

NC = 2
NS = 16
NW = NC * NS


NB = 4


def _sc_bag(xf, didx, zeros, embed, batch):
    rtot, lw = xf.shape
    d = embed.shape[1]
    rpw = rtot // NW
    bpw = batch // NW
    mesh = plsc.VectorSubcoreMesh(core_axis_name="c", subcore_axis_name="s")

    @functools.partial(
        pl.kernel,
        out_type=jax.ShapeDtypeStruct((batch, d), jnp.float32),
        mesh=mesh,
        scratch_types=[
            pltpu.VMEM((rpw, lw), jnp.int32),
            pltpu.VMEM((rpw, lw), jnp.int32),
            [pltpu.VMEM((lw, d), jnp.float32) for _ in range(NB)],
            pltpu.VMEM((bpw, d), jnp.float32),
            pltpu.VMEM_SHARED((batch, d), jnp.float32),
            [pltpu.SemaphoreType.DMA for _ in range(NB)],
            [pltpu.SemaphoreType.DMA for _ in range(NB)],
        ],
        compiler_params=pltpu.CompilerParams(use_tc_tiling_on_sc=False),
    )
    def k(xf_h, didx_h, zeros_h, embed_h, out_h,
          xblk, dblk, gbs, obuf, acc, gsems, ssems):
        c = lax.axis_index("c")
        s = lax.axis_index("s")
        wid = s * NC + c
        rbase = wid * rpw
        bbase = wid * bpw

        pltpu.sync_copy(xf_h.at[pl.ds(rbase, rpw)], xblk)
        pltpu.sync_copy(didx_h.at[pl.ds(rbase, rpw)], dblk)
        pltpu.sync_copy(zeros_h, obuf)
        pltpu.sync_copy(obuf, acc.at[pl.ds(bbase, bpw)])

        for b in range(NB):
            pltpu.async_copy(embed_h.at[xblk.at[b]], gbs[b], gsems[b])

        def body(i, carry):
            g = NB * i
            for b in range(NB):
                pltpu.make_async_copy(embed_h.at[xblk.at[0]], gbs[b],
                                      gsems[b]).wait()
                pltpu.async_copy(gbs[b], acc.at[dblk.at[g + b]], ssems[b],
                                 add=True)
            for b in range(NB):
                pltpu.make_async_copy(gbs[b], acc.at[dblk.at[0]],
                                      ssems[b]).wait()
                pltpu.async_copy(embed_h.at[xblk.at[g + NB + b]], gbs[b],
                                 gsems[b])
            return carry

        lax.fori_loop(0, rpw // NB - 1, body, 0)

        g_last = rpw - NB
        for b in range(NB):
            pltpu.make_async_copy(embed_h.at[xblk.at[0]], gbs[b],
                                  gsems[b]).wait()
            pltpu.async_copy(gbs[b], acc.at[dblk.at[g_last + b]], ssems[b],
                             add=True)
        for b in range(NB):
            pltpu.make_async_copy(gbs[b], acc.at[dblk.at[0]], ssems[b]).wait()

        pltpu.sync_copy(acc.at[pl.ds(bbase, bpw)], obuf)
        pltpu.sync_copy(obuf, out_h.at[pl.ds(bbase, bpw)])

    return k(xf, didx, zeros, embed)


def _tc_mlp(s, W1, b1, W2, b2, n_rows):
    batch, d = s.shape
    hid = W1.shape[0]
    ncls = W2.shape[0]
    hp = 128
    W1p = jnp.zeros((hp, d), jnp.float32).at[:hid].set(W1)
    b1p = jnp.zeros((1, hp), jnp.float32).at[0, :hid].set(b1)
    W2p = jnp.zeros((ncls, hp), jnp.float32).at[:, :hid].set(W2)
    b2p = b2.reshape(1, ncls)
    bt = 512
    scale = 1.0 / n_rows

    def body(s_ref, w1_ref, b1_ref, w2_ref, b2_ref, o_ref):
        m = s_ref[...] * scale
        h = lax.dot_general(m, w1_ref[...], (((1,), (1,)), ((), ())),
                            preferred_element_type=jnp.float32)
        h = jnp.maximum(h + b1_ref[...], 0.0)
        o = lax.dot_general(h, w2_ref[...], (((1,), (1,)), ((), ())),
                            preferred_element_type=jnp.float32)
        o_ref[...] = o + b2_ref[...]

    return pl.pallas_call(
        body,
        grid=(batch // bt,),
        in_specs=[
            pl.BlockSpec((bt, d), lambda i: (i, 0)),
            pl.BlockSpec((hp, d), lambda i: (0, 0)),
            pl.BlockSpec((1, hp), lambda i: (0, 0)),
            pl.BlockSpec((ncls, hp), lambda i: (0, 0)),
            pl.BlockSpec((1, ncls), lambda i: (0, 0)),
        ],
        out_specs=pl.BlockSpec((bt, ncls), lambda i: (i, 0)),
        out_shape=jax.ShapeDtypeStruct((batch, ncls), jnp.float32),
    )(s, W1p, b1p, W2p, b2p)


def kernel(x, embed, W1, b1, W2, b2):
    batch, hist = x.shape
    d = embed.shape[1]
    lw = 128
    xf = x.reshape(-1, lw)
    didx = (jnp.arange(batch * hist, dtype=jnp.int32) // hist).reshape(-1, lw)
    zeros = jnp.zeros((batch // NW, d), jnp.float32)
    sums = _sc_bag(xf, didx, zeros, embed, batch)
    return _tc_mlp(sums, W1, b1, W2, b2, batch)
